# Initial kernel scaffold; baseline (speedup 1.0000x reference)
import jax
import jax.numpy as jnp
from jax.experimental import pallas as pl


def kernel(x, edge_index, batch, shape_emb, color_emb, W_node, b_node, W1_rel, b1_rel, W1_root, W2_rel, b2_rel, W2_root, W_cls, b_cls):
    raise NotImplementedError("write your pallas kernel here")


# speedup vs baseline: 5.2125x; 5.2125x over previous
import functools

import jax
import jax.numpy as jnp
from jax import lax
from jax.experimental import pallas as pl
from jax.experimental.pallas import tpu as pltpu
from jax.experimental.pallas import tpu_sc as plsc

N = 50000
E = 800000
G = 256
EP = 819200
CHUNK = 128
NSUB = 16
NACC = 50048
R = 2000
GRID = N // R


def _node_mlp_body(x0_ref, x1_ref, se_ref, ce_ref, wn1_ref, wn2_ref, b_ref,
                   h0a_ref, h0b_ref):
    tab_s = jnp.dot(se_ref[...], wn1_ref[...], preferred_element_type=jnp.float32,
                    precision=lax.Precision.HIGHEST)
    tab_c = jnp.dot(ce_ref[...], wn2_ref[...], preferred_element_type=jnp.float32,
                    precision=lax.Precision.HIGHEST)
    tab = jnp.concatenate([tab_s, tab_c], axis=0)
    x0 = jnp.reshape(x0_ref[...], (1, R))
    x1 = jnp.reshape(x1_ref[...], (1, R))
    j = lax.broadcasted_iota(jnp.int32, (32, R), 0)
    tgt = jnp.where(j < 16, x0, x1 + 16)
    oht = (j == tgt).astype(jnp.float32)
    h0 = lax.dot_general(oht, tab, (((0,), (0,)), ((), ())),
                         preferred_element_type=jnp.float32,
                         precision=lax.Precision.HIGHEST)
    h0 = jnp.maximum(h0 + b_ref[...], 0.0)
    h0a_ref[...] = h0[:, :16]
    h0b_ref[...] = h0[:, 16:]


def _node_mlp(x0, x1, shape_emb, color_emb, wn1t, wn2t, b_node):
    return pl.pallas_call(
        _node_mlp_body,
        grid=(GRID,),
        in_specs=[
            pl.BlockSpec((1, 1, R), lambda i: (i, 0, 0)),
            pl.BlockSpec((1, 1, R), lambda i: (i, 0, 0)),
            pl.BlockSpec((16, 8), lambda i: (0, 0)),
            pl.BlockSpec((16, 8), lambda i: (0, 0)),
            pl.BlockSpec((8, 32), lambda i: (0, 0)),
            pl.BlockSpec((8, 32), lambda i: (0, 0)),
            pl.BlockSpec((1, 32), lambda i: (0, 0)),
        ],
        out_specs=[
            pl.BlockSpec((R, 16), lambda i: (i, 0)),
            pl.BlockSpec((R, 16), lambda i: (i, 0)),
        ],
        out_shape=[
            jax.ShapeDtypeStruct((N, 16), jnp.float32),
            jax.ShapeDtypeStruct((N, 16), jnp.float32),
        ],
    )(x0, x1, shape_emb, color_emb, wn1t, wn2t, b_node)


def _conv_lin_body(a_ref, b_ref, c_ref, d_ref, wc_ref, bias_ref, o1_ref, o2_ref):
    cat = jnp.concatenate(
        [a_ref[...], b_ref[...], c_ref[...], d_ref[...]], axis=1)
    h = jnp.dot(cat, wc_ref[...], preferred_element_type=jnp.float32,
                precision=lax.Precision.HIGHEST)
    h = jnp.maximum(h + bias_ref[...], 0.0)
    o1_ref[...] = h[:, :32]
    o2_ref[...] = h[:, 32:]


def _conv1_lin(a1a, a1b, h0a, h0b, wc, bias):
    return pl.pallas_call(
        _conv_lin_body,
        grid=(GRID,),
        in_specs=[
            pl.BlockSpec((R, 16), lambda i: (i, 0)),
            pl.BlockSpec((R, 16), lambda i: (i, 0)),
            pl.BlockSpec((R, 16), lambda i: (i, 0)),
            pl.BlockSpec((R, 16), lambda i: (i, 0)),
            pl.BlockSpec((64, 64), lambda i: (0, 0)),
            pl.BlockSpec((1, 64), lambda i: (0, 0)),
        ],
        out_specs=[
            pl.BlockSpec((R, 32), lambda i: (i, 0)),
            pl.BlockSpec((R, 32), lambda i: (i, 0)),
        ],
        out_shape=[
            jax.ShapeDtypeStruct((N, 32), jnp.float32),
            jax.ShapeDtypeStruct((N, 32), jnp.float32),
        ],
    )(a1a, a1b, h0a, h0b, wc, bias)


def _final_body(a_ref, b_ref, c_ref, d_ref, wc_ref, bias_ref, batch_ref,
                wcls_ref, bcls_ref, out_ref, sums_ref, cnt_ref):
    i = pl.program_id(0)
    cat = jnp.concatenate(
        [a_ref[...], b_ref[...], c_ref[...], d_ref[...]], axis=1)
    h2 = jnp.dot(cat, wc_ref[...], preferred_element_type=jnp.float32,
                 precision=lax.Precision.HIGHEST)
    h2 = jnp.maximum(h2 + bias_ref[...], 0.0)
    brow = jnp.reshape(batch_ref[...], (1, R))
    gi = lax.broadcasted_iota(jnp.int32, (G, R), 0)
    oht = (gi == brow).astype(jnp.float32)
    psum = jnp.dot(oht, h2, preferred_element_type=jnp.float32,
                   precision=lax.Precision.HIGHEST)
    pcnt = jnp.sum(oht, axis=1, keepdims=True)

    @pl.when(i == 0)
    def _init():
        sums_ref[...] = psum
        cnt_ref[...] = pcnt

    @pl.when(i > 0)
    def _acc():
        sums_ref[...] += psum
        cnt_ref[...] += pcnt

    @pl.when(i == GRID - 1)
    def _fin():
        logits = jnp.dot(sums_ref[...], wcls_ref[...],
                         preferred_element_type=jnp.float32,
                         precision=lax.Precision.HIGHEST)
        out_ref[...] = logits / jnp.maximum(cnt_ref[...], 1.0) + bcls_ref[...]


def _final(a2a, a2b, h1a, h1b, wc, bias, batch3, wclst, bcls):
    return pl.pallas_call(
        _final_body,
        grid=(GRID,),
        in_specs=[
            pl.BlockSpec((R, 32), lambda i: (i, 0)),
            pl.BlockSpec((R, 32), lambda i: (i, 0)),
            pl.BlockSpec((R, 32), lambda i: (i, 0)),
            pl.BlockSpec((R, 32), lambda i: (i, 0)),
            pl.BlockSpec((128, 64), lambda i: (0, 0)),
            pl.BlockSpec((1, 64), lambda i: (0, 0)),
            pl.BlockSpec((1, 1, R), lambda i: (i, 0, 0)),
            pl.BlockSpec((64, 10), lambda i: (0, 0)),
            pl.BlockSpec((1, 10), lambda i: (0, 0)),
        ],
        out_specs=pl.BlockSpec((G, 10), lambda i: (0, 0)),
        out_shape=jax.ShapeDtypeStruct((G, 10), jnp.float32),
        scratch_shapes=[
            pltpu.VMEM((G, 64), jnp.float32),
            pltpu.VMEM((G, 1), jnp.float32),
        ],
    )(a2a, a2b, h1a, h1b, wc, bias, batch3, wclst, bcls)


@functools.cache
def _make_sc_agg(F2):
    eps = EP // NSUB
    nchunk = eps // CHUNK
    ib = 16
    nblk = nchunk // ib
    zr = NACC // NSUB
    wr = 3128
    wr_last = N - 15 * wr

    mesh = plsc.VectorSubcoreMesh(core_axis_name="c", subcore_axis_name="s",
                                  num_cores=2, num_subcores=NSUB)

    def body(ha, hb, srcm, dstm, zeros, aa, ab, srcbuf, dstbuf, rows, acc, sem):
        cid = lax.axis_index("c")
        sid = lax.axis_index("s")

        def run(h_hbm, out_hbm):
            pltpu.sync_copy(zeros, acc.at[pl.ds(sid * zr, zr)])
            plsc.subcore_barrier()

            row0 = sid * nchunk

            def blk(b, carry):
                r0 = row0 + b * ib
                pltpu.sync_copy(srcm.at[pl.ds(r0, ib)], srcbuf)
                pltpu.sync_copy(dstm.at[pl.ds(r0, ib)], dstbuf)
                for j in range(ib):
                    pltpu.async_copy(h_hbm.at[srcbuf.at[j]], rows, sem).wait()
                    pltpu.sync_copy(rows, acc.at[dstbuf.at[j]], add=True)
                return carry

            lax.fori_loop(0, nblk, blk, 0)
            plsc.subcore_barrier()

            @pl.when(sid < NSUB - 1)
            def _wb():
                pltpu.sync_copy(acc.at[pl.ds(sid * wr, wr)],
                                out_hbm.at[pl.ds(sid * wr, wr)])

            @pl.when(sid == NSUB - 1)
            def _wb_last():
                pltpu.sync_copy(acc.at[pl.ds((NSUB - 1) * wr, wr_last)],
                                out_hbm.at[pl.ds((NSUB - 1) * wr, wr_last)])

        @pl.when(cid == 0)
        def _c0():
            run(ha, aa)

        @pl.when(cid == 1)
        def _c1():
            run(hb, ab)

    return pl.kernel(
        body,
        out_type=[
            jax.ShapeDtypeStruct((N, F2), jnp.float32),
            jax.ShapeDtypeStruct((N, F2), jnp.float32),
        ],
        mesh=mesh,
        compiler_params=pltpu.CompilerParams(use_tc_tiling_on_sc=False),
        scratch_types=[
            pltpu.VMEM((ib, CHUNK), jnp.int32),
            pltpu.VMEM((ib, CHUNK), jnp.int32),
            pltpu.VMEM((CHUNK, F2), jnp.float32),
            pltpu.VMEM_SHARED((NACC, F2), jnp.float32),
            pltpu.SemaphoreType.DMA,
        ],
    )


def kernel(x, edge_index, batch, shape_emb, color_emb, W_node, b_node,
           W1_rel, b1_rel, W1_root, W2_rel, b2_rel, W2_root, W_cls, b_cls):
    x0 = x[:, 0].reshape(GRID, 1, R).astype(jnp.int32)
    x1 = x[:, 1].reshape(GRID, 1, R).astype(jnp.int32)
    batch3 = batch.reshape(GRID, 1, R).astype(jnp.int32)

    src = edge_index[0].astype(jnp.int32)
    dst = edge_index[1].astype(jnp.int32)
    pad = EP - E
    pad_dst = N + (jnp.arange(pad, dtype=jnp.int32) % (NACC - N))
    srcm = jnp.concatenate([src, jnp.zeros((pad,), jnp.int32)]).reshape(EP // CHUNK, CHUNK)
    dstm = jnp.concatenate([dst, pad_dst]).reshape(EP // CHUNK, CHUNK)

    zr = NACC // NSUB
    z16 = jnp.zeros((zr, 16), jnp.float32)
    z32 = jnp.zeros((zr, 32), jnp.float32)

    wn1t = W_node[:, :8].T
    wn2t = W_node[:, 8:].T
    bn = b_node.reshape(1, 32)
    wc1 = jnp.concatenate([W1_rel.T, W1_root.T], axis=0)
    b1 = b1_rel.reshape(1, 64)
    wc2 = jnp.concatenate([W2_rel.T, W2_root.T], axis=0)
    b2 = b2_rel.reshape(1, 64)
    wclst = W_cls.T
    bcls = b_cls.reshape(1, 10)

    h0a, h0b = _node_mlp(x0, x1, shape_emb, color_emb, wn1t, wn2t, bn)
    a1a, a1b = _make_sc_agg(16)(h0a, h0b, srcm, dstm, z16)
    h1a, h1b = _conv1_lin(a1a, a1b, h0a, h0b, wc1, b1)
    a2a, a2b = _make_sc_agg(32)(h1a, h1b, srcm, dstm, z32)
    return _final(a2a, a2b, h1a, h1b, wc2, b2, batch3, wclst, bcls)


# speedup vs baseline: 7.7639x; 1.4895x over previous
import functools

import jax
import jax.numpy as jnp
from jax import lax
from jax.experimental import pallas as pl
from jax.experimental.pallas import tpu as pltpu
from jax.experimental.pallas import tpu_sc as plsc

N = 50000
E = 800000
G = 256
EP = 819200
CHUNK = 128
NSUB = 16
NACC = 50048
R = 2000
GRID = N // R


def _node_mlp_body(x0_ref, x1_ref, se_ref, ce_ref, wn1_ref, wn2_ref, b_ref,
                   h0a_ref, h0b_ref):
    tab_s = jnp.dot(se_ref[...], wn1_ref[...], preferred_element_type=jnp.float32,
                    precision=lax.Precision.HIGHEST)
    tab_c = jnp.dot(ce_ref[...], wn2_ref[...], preferred_element_type=jnp.float32,
                    precision=lax.Precision.HIGHEST)
    tab = jnp.concatenate([tab_s, tab_c], axis=0)
    x0 = jnp.reshape(x0_ref[...], (1, R))
    x1 = jnp.reshape(x1_ref[...], (1, R))
    j = lax.broadcasted_iota(jnp.int32, (32, R), 0)
    tgt = jnp.where(j < 16, x0, x1 + 16)
    oht = (j == tgt).astype(jnp.float32)
    h0 = lax.dot_general(oht, tab, (((0,), (0,)), ((), ())),
                         preferred_element_type=jnp.float32,
                         precision=lax.Precision.HIGHEST)
    h0 = jnp.maximum(h0 + b_ref[...], 0.0)
    h0a_ref[...] = h0[:, :16]
    h0b_ref[...] = h0[:, 16:]


def _node_mlp(x0, x1, shape_emb, color_emb, wn1t, wn2t, b_node):
    return pl.pallas_call(
        _node_mlp_body,
        grid=(GRID,),
        in_specs=[
            pl.BlockSpec((1, 1, R), lambda i: (i, 0, 0)),
            pl.BlockSpec((1, 1, R), lambda i: (i, 0, 0)),
            pl.BlockSpec((16, 8), lambda i: (0, 0)),
            pl.BlockSpec((16, 8), lambda i: (0, 0)),
            pl.BlockSpec((8, 32), lambda i: (0, 0)),
            pl.BlockSpec((8, 32), lambda i: (0, 0)),
            pl.BlockSpec((1, 32), lambda i: (0, 0)),
        ],
        out_specs=[
            pl.BlockSpec((R, 16), lambda i: (i, 0)),
            pl.BlockSpec((R, 16), lambda i: (i, 0)),
        ],
        out_shape=[
            jax.ShapeDtypeStruct((N, 16), jnp.float32),
            jax.ShapeDtypeStruct((N, 16), jnp.float32),
        ],
    )(x0, x1, shape_emb, color_emb, wn1t, wn2t, b_node)


def _conv_lin_body(a_ref, b_ref, c_ref, d_ref, wc_ref, bias_ref, o1_ref, o2_ref):
    cat = jnp.concatenate(
        [a_ref[...], b_ref[...], c_ref[...], d_ref[...]], axis=1)
    h = jnp.dot(cat, wc_ref[...], preferred_element_type=jnp.float32,
                precision=lax.Precision.HIGHEST)
    h = jnp.maximum(h + bias_ref[...], 0.0)
    o1_ref[...] = h[:, :32]
    o2_ref[...] = h[:, 32:]


def _conv1_lin(a1a, a1b, h0a, h0b, wc, bias):
    return pl.pallas_call(
        _conv_lin_body,
        grid=(GRID,),
        in_specs=[
            pl.BlockSpec((R, 16), lambda i: (i, 0)),
            pl.BlockSpec((R, 16), lambda i: (i, 0)),
            pl.BlockSpec((R, 16), lambda i: (i, 0)),
            pl.BlockSpec((R, 16), lambda i: (i, 0)),
            pl.BlockSpec((64, 64), lambda i: (0, 0)),
            pl.BlockSpec((1, 64), lambda i: (0, 0)),
        ],
        out_specs=[
            pl.BlockSpec((R, 32), lambda i: (i, 0)),
            pl.BlockSpec((R, 32), lambda i: (i, 0)),
        ],
        out_shape=[
            jax.ShapeDtypeStruct((N, 32), jnp.float32),
            jax.ShapeDtypeStruct((N, 32), jnp.float32),
        ],
    )(a1a, a1b, h0a, h0b, wc, bias)


def _final_body(a_ref, b_ref, c_ref, d_ref, wc_ref, bias_ref, batch_ref,
                wcls_ref, bcls_ref, out_ref, sums_ref, cnt_ref):
    i = pl.program_id(0)
    cat = jnp.concatenate(
        [a_ref[...], b_ref[...], c_ref[...], d_ref[...]], axis=1)
    h2 = jnp.dot(cat, wc_ref[...], preferred_element_type=jnp.float32,
                 precision=lax.Precision.HIGHEST)
    h2 = jnp.maximum(h2 + bias_ref[...], 0.0)
    brow = jnp.reshape(batch_ref[...], (1, R))
    gi = lax.broadcasted_iota(jnp.int32, (G, R), 0)
    oht = (gi == brow).astype(jnp.float32)
    psum = jnp.dot(oht, h2, preferred_element_type=jnp.float32,
                   precision=lax.Precision.HIGHEST)
    pcnt = jnp.sum(oht, axis=1, keepdims=True)

    @pl.when(i == 0)
    def _init():
        sums_ref[...] = psum
        cnt_ref[...] = pcnt

    @pl.when(i > 0)
    def _acc():
        sums_ref[...] += psum
        cnt_ref[...] += pcnt

    @pl.when(i == GRID - 1)
    def _fin():
        logits = jnp.dot(sums_ref[...], wcls_ref[...],
                         preferred_element_type=jnp.float32,
                         precision=lax.Precision.HIGHEST)
        out_ref[...] = logits / jnp.maximum(cnt_ref[...], 1.0) + bcls_ref[...]


def _final(a2a, a2b, h1a, h1b, wc, bias, batch3, wclst, bcls):
    return pl.pallas_call(
        _final_body,
        grid=(GRID,),
        in_specs=[
            pl.BlockSpec((R, 32), lambda i: (i, 0)),
            pl.BlockSpec((R, 32), lambda i: (i, 0)),
            pl.BlockSpec((R, 32), lambda i: (i, 0)),
            pl.BlockSpec((R, 32), lambda i: (i, 0)),
            pl.BlockSpec((128, 64), lambda i: (0, 0)),
            pl.BlockSpec((1, 64), lambda i: (0, 0)),
            pl.BlockSpec((1, 1, R), lambda i: (i, 0, 0)),
            pl.BlockSpec((64, 10), lambda i: (0, 0)),
            pl.BlockSpec((1, 10), lambda i: (0, 0)),
        ],
        out_specs=pl.BlockSpec((G, 10), lambda i: (0, 0)),
        out_shape=jax.ShapeDtypeStruct((G, 10), jnp.float32),
        scratch_shapes=[
            pltpu.VMEM((G, 64), jnp.float32),
            pltpu.VMEM((G, 1), jnp.float32),
        ],
    )(a2a, a2b, h1a, h1b, wc, bias, batch3, wclst, bcls)


@functools.cache
def _make_sc_agg(F2):
    eps = EP // NSUB
    nchunk = eps // CHUNK
    ib = 16
    nblk = nchunk // ib
    zr = NACC // NSUB
    wr = 3128
    wr_last = N - 15 * wr
    D = 4

    mesh = plsc.VectorSubcoreMesh(core_axis_name="c", subcore_axis_name="s",
                                  num_cores=2, num_subcores=NSUB)

    def body(ha, hb, srcm, dstm, zeros, aa, ab, srcbuf, dstbuf, rows, acc,
             g0, g1, g2, g3):
        cid = lax.axis_index("c")
        sid = lax.axis_index("s")
        gsem = (g0, g1, g2, g3)

        def run(h_hbm, out_hbm):
            pltpu.sync_copy(zeros, acc.at[pl.ds(sid * zr, zr)])
            plsc.subcore_barrier()

            row0 = sid * nchunk

            def gather(islot, j, rslot):
                pltpu.async_copy(h_hbm.at[srcbuf.at[islot, j]],
                                 rows.at[rslot], gsem[rslot])

            def scatter(islot, j, rslot):
                pltpu.make_async_copy(h_hbm.at[srcbuf.at[islot, j]],
                                      rows.at[rslot], gsem[rslot]).wait()
                pltpu.sync_copy(rows.at[rslot], acc.at[dstbuf.at[islot, j]],
                                add=True)

            pltpu.sync_copy(srcm.at[pl.ds(row0, ib)], srcbuf.at[0])
            pltpu.sync_copy(dstm.at[pl.ds(row0, ib)], dstbuf.at[0])
            for j in range(D):
                gather(0, j, j % D)

            def blk(b, carry):
                islot = lax.rem(b, 2)
                nslot = lax.rem(b + 1, 2)
                r0 = row0 + (b + 1) * ib
                pltpu.sync_copy(srcm.at[pl.ds(r0, ib)], srcbuf.at[nslot])
                pltpu.sync_copy(dstm.at[pl.ds(r0, ib)], dstbuf.at[nslot])
                for j in range(ib):
                    scatter(islot, j, j % D)
                    jf = (j + D) % ib
                    gather(islot if j + D < ib else nslot, jf, j % D)
                return carry

            lax.fori_loop(0, nblk - 1, blk, 0)

            for j in range(ib):
                scatter(0, j, j % D)
                if j + D < ib:
                    gather(0, j + D, j % D)

            plsc.subcore_barrier()

            @pl.when(sid < NSUB - 1)
            def _wb():
                pltpu.sync_copy(acc.at[pl.ds(sid * wr, wr)],
                                out_hbm.at[pl.ds(sid * wr, wr)])

            @pl.when(sid == NSUB - 1)
            def _wb_last():
                pltpu.sync_copy(acc.at[pl.ds((NSUB - 1) * wr, wr_last)],
                                out_hbm.at[pl.ds((NSUB - 1) * wr, wr_last)])

        @pl.when(cid == 0)
        def _c0():
            run(ha, aa)

        @pl.when(cid == 1)
        def _c1():
            run(hb, ab)

    return pl.kernel(
        body,
        out_type=[
            jax.ShapeDtypeStruct((N, F2), jnp.float32),
            jax.ShapeDtypeStruct((N, F2), jnp.float32),
        ],
        mesh=mesh,
        compiler_params=pltpu.CompilerParams(use_tc_tiling_on_sc=False),
        scratch_types=[
            pltpu.VMEM((2, ib, CHUNK), jnp.int32),
            pltpu.VMEM((2, ib, CHUNK), jnp.int32),
            pltpu.VMEM((4, CHUNK, F2), jnp.float32),
            pltpu.VMEM_SHARED((NACC, F2), jnp.float32),
            pltpu.SemaphoreType.DMA,
            pltpu.SemaphoreType.DMA,
            pltpu.SemaphoreType.DMA,
            pltpu.SemaphoreType.DMA,
        ],
    )


def kernel(x, edge_index, batch, shape_emb, color_emb, W_node, b_node,
           W1_rel, b1_rel, W1_root, W2_rel, b2_rel, W2_root, W_cls, b_cls):
    x0 = x[:, 0].reshape(GRID, 1, R).astype(jnp.int32)
    x1 = x[:, 1].reshape(GRID, 1, R).astype(jnp.int32)
    batch3 = batch.reshape(GRID, 1, R).astype(jnp.int32)

    src = edge_index[0].astype(jnp.int32)
    dst = edge_index[1].astype(jnp.int32)
    pad = EP - E
    pad_dst = N + (jnp.arange(pad, dtype=jnp.int32) % (NACC - N))
    srcm = jnp.concatenate([src, jnp.zeros((pad,), jnp.int32)]).reshape(EP // CHUNK, CHUNK)
    dstm = jnp.concatenate([dst, pad_dst]).reshape(EP // CHUNK, CHUNK)

    zr = NACC // NSUB
    z16 = jnp.zeros((zr, 16), jnp.float32)
    z32 = jnp.zeros((zr, 32), jnp.float32)

    wn1t = W_node[:, :8].T
    wn2t = W_node[:, 8:].T
    bn = b_node.reshape(1, 32)
    wc1 = jnp.concatenate([W1_rel.T, W1_root.T], axis=0)
    b1 = b1_rel.reshape(1, 64)
    wc2 = jnp.concatenate([W2_rel.T, W2_root.T], axis=0)
    b2 = b2_rel.reshape(1, 64)
    wclst = W_cls.T
    bcls = b_cls.reshape(1, 10)

    h0a, h0b = _node_mlp(x0, x1, shape_emb, color_emb, wn1t, wn2t, bn)
    a1a, a1b = _make_sc_agg(16)(h0a, h0b, srcm, dstm, z16)
    h1a, h1b = _conv1_lin(a1a, a1b, h0a, h0b, wc1, b1)
    a2a, a2b = _make_sc_agg(32)(h1a, h1b, srcm, dstm, z32)
    return _final(a2a, a2b, h1a, h1b, wc2, b2, batch3, wclst, bcls)


# speedup vs baseline: 9.2432x; 1.1905x over previous
import functools

import jax
import jax.numpy as jnp
from jax import lax
from jax.experimental import pallas as pl
from jax.experimental.pallas import tpu as pltpu
from jax.experimental.pallas import tpu_sc as plsc

N = 50000
E = 800000
G = 256
EP = 819200
CHUNK = 128
NSUB = 16
NACC = 50048
FS = 16
R = 2000
GRID = N // R


def _node_mlp_body(x0_ref, x1_ref, se_ref, ce_ref, wn1_ref, wn2_ref, b_ref,
                   h0a_ref, h0b_ref):
    tab_s = jnp.dot(se_ref[...], wn1_ref[...], preferred_element_type=jnp.float32,
                    precision=lax.Precision.HIGHEST)
    tab_c = jnp.dot(ce_ref[...], wn2_ref[...], preferred_element_type=jnp.float32,
                    precision=lax.Precision.HIGHEST)
    tab = jnp.concatenate([tab_s, tab_c], axis=0)
    x0 = jnp.reshape(x0_ref[...], (1, R))
    x1 = jnp.reshape(x1_ref[...], (1, R))
    j = lax.broadcasted_iota(jnp.int32, (32, R), 0)
    tgt = jnp.where(j < 16, x0, x1 + 16)
    oht = (j == tgt).astype(jnp.float32)
    h0 = lax.dot_general(oht, tab, (((0,), (0,)), ((), ())),
                         preferred_element_type=jnp.float32,
                         precision=lax.Precision.HIGHEST)
    h0 = jnp.maximum(h0 + b_ref[...], 0.0)
    h0a_ref[...] = h0[:, :16]
    h0b_ref[...] = h0[:, 16:]


def _node_mlp(x0, x1, shape_emb, color_emb, wn1t, wn2t, b_node):
    return pl.pallas_call(
        _node_mlp_body,
        grid=(GRID,),
        in_specs=[
            pl.BlockSpec((1, 1, R), lambda i: (i, 0, 0)),
            pl.BlockSpec((1, 1, R), lambda i: (i, 0, 0)),
            pl.BlockSpec((16, 8), lambda i: (0, 0)),
            pl.BlockSpec((16, 8), lambda i: (0, 0)),
            pl.BlockSpec((8, 32), lambda i: (0, 0)),
            pl.BlockSpec((8, 32), lambda i: (0, 0)),
            pl.BlockSpec((1, 32), lambda i: (0, 0)),
        ],
        out_specs=[
            pl.BlockSpec((R, 16), lambda i: (i, 0)),
            pl.BlockSpec((R, 16), lambda i: (i, 0)),
        ],
        out_shape=[
            jax.ShapeDtypeStruct((N, 16), jnp.float32),
            jax.ShapeDtypeStruct((N, 16), jnp.float32),
        ],
    )(x0, x1, shape_emb, color_emb, wn1t, wn2t, b_node)


def _conv_lin_body(a_ref, b_ref, c_ref, d_ref, wc_ref, bias_ref,
                   o1_ref, o2_ref, o3_ref, o4_ref):
    cat = jnp.concatenate(
        [a_ref[...], b_ref[...], c_ref[...], d_ref[...]], axis=1)
    h = jnp.dot(cat, wc_ref[...], preferred_element_type=jnp.float32,
                precision=lax.Precision.HIGHEST)
    h = jnp.maximum(h + bias_ref[...], 0.0)
    o1_ref[...] = h[:, :16]
    o2_ref[...] = h[:, 16:32]
    o3_ref[...] = h[:, 32:48]
    o4_ref[...] = h[:, 48:]


def _conv1_lin(a1a, a1b, h0a, h0b, wc, bias):
    return pl.pallas_call(
        _conv_lin_body,
        grid=(GRID,),
        in_specs=[
            pl.BlockSpec((R, 16), lambda i: (i, 0)),
            pl.BlockSpec((R, 16), lambda i: (i, 0)),
            pl.BlockSpec((R, 16), lambda i: (i, 0)),
            pl.BlockSpec((R, 16), lambda i: (i, 0)),
            pl.BlockSpec((64, 64), lambda i: (0, 0)),
            pl.BlockSpec((1, 64), lambda i: (0, 0)),
        ],
        out_specs=[pl.BlockSpec((R, 16), lambda i: (i, 0))] * 4,
        out_shape=[jax.ShapeDtypeStruct((N, 16), jnp.float32)] * 4,
    )(a1a, a1b, h0a, h0b, wc, bias)


def _final_body(a_ref, b_ref, c_ref, d_ref, e_ref, f_ref, g_ref, hh_ref,
                wc_ref, bias_ref, batch_ref,
                wcls_ref, bcls_ref, out_ref, sums_ref, cnt_ref):
    i = pl.program_id(0)
    cat = jnp.concatenate(
        [a_ref[...], b_ref[...], c_ref[...], d_ref[...],
         e_ref[...], f_ref[...], g_ref[...], hh_ref[...]], axis=1)
    h2 = jnp.dot(cat, wc_ref[...], preferred_element_type=jnp.float32,
                 precision=lax.Precision.HIGHEST)
    h2 = jnp.maximum(h2 + bias_ref[...], 0.0)
    brow = jnp.reshape(batch_ref[...], (1, R))
    gi = lax.broadcasted_iota(jnp.int32, (G, R), 0)
    oht = (gi == brow).astype(jnp.float32)
    psum = jnp.dot(oht, h2, preferred_element_type=jnp.float32,
                   precision=lax.Precision.HIGHEST)
    pcnt = jnp.sum(oht, axis=1, keepdims=True)

    @pl.when(i == 0)
    def _init():
        sums_ref[...] = psum
        cnt_ref[...] = pcnt

    @pl.when(i > 0)
    def _acc():
        sums_ref[...] += psum
        cnt_ref[...] += pcnt

    @pl.when(i == GRID - 1)
    def _fin():
        logits = jnp.dot(sums_ref[...], wcls_ref[...],
                         preferred_element_type=jnp.float32,
                         precision=lax.Precision.HIGHEST)
        out_ref[...] = logits / jnp.maximum(cnt_ref[...], 1.0) + bcls_ref[...]


def _final(a2, h1, wc, bias, batch3, wclst, bcls):
    return pl.pallas_call(
        _final_body,
        grid=(GRID,),
        in_specs=[pl.BlockSpec((R, 16), lambda i: (i, 0))] * 8 + [
            pl.BlockSpec((128, 64), lambda i: (0, 0)),
            pl.BlockSpec((1, 64), lambda i: (0, 0)),
            pl.BlockSpec((1, 1, R), lambda i: (i, 0, 0)),
            pl.BlockSpec((64, 10), lambda i: (0, 0)),
            pl.BlockSpec((1, 10), lambda i: (0, 0)),
        ],
        out_specs=pl.BlockSpec((G, 10), lambda i: (0, 0)),
        out_shape=jax.ShapeDtypeStruct((G, 10), jnp.float32),
        scratch_shapes=[
            pltpu.VMEM((G, 64), jnp.float32),
            pltpu.VMEM((G, 1), jnp.float32),
        ],
    )(*a2, *h1, wc, bias, batch3, wclst, bcls)


@functools.cache
def _make_sc_agg(npc):
    eps = EP // NSUB
    nchunk = eps // CHUNK
    ib = 16
    nblk = nchunk // ib
    zr = NACC // NSUB
    wr = 3128
    wr_last = N - 15 * wr
    D = 4

    mesh = plsc.VectorSubcoreMesh(core_axis_name="c", subcore_axis_name="s",
                                  num_cores=2, num_subcores=NSUB)

    def body(*refs):
        hs = refs[:2 * npc]
        srcm, dstm, zeros = refs[2 * npc:2 * npc + 3]
        outs = refs[2 * npc + 3:4 * npc + 3]
        (srcbuf, dstbuf, rows, acc, h_sp,
         g0, g1, g2, g3) = refs[4 * npc + 3:]
        cid = lax.axis_index("c")
        sid = lax.axis_index("s")
        gsem = (g0, g1, g2, g3)

        def run(h_hbm, out_hbm):
            pltpu.sync_copy(zeros, acc.at[pl.ds(sid * zr, zr)])

            @pl.when(sid < NSUB - 1)
            def _ld():
                pltpu.sync_copy(h_hbm.at[pl.ds(sid * wr, wr)],
                                h_sp.at[pl.ds(sid * wr, wr)])

            @pl.when(sid == NSUB - 1)
            def _ld_last():
                pltpu.sync_copy(h_hbm.at[pl.ds((NSUB - 1) * wr, wr_last)],
                                h_sp.at[pl.ds((NSUB - 1) * wr, wr_last)])

            plsc.subcore_barrier()

            row0 = sid * nchunk

            def gather(islot, j, rslot):
                pltpu.async_copy(h_sp.at[srcbuf.at[islot, j]],
                                 rows.at[rslot], gsem[rslot])

            def scatter(islot, j, rslot):
                pltpu.make_async_copy(h_sp.at[srcbuf.at[islot, j]],
                                      rows.at[rslot], gsem[rslot]).wait()
                pltpu.sync_copy(rows.at[rslot], acc.at[dstbuf.at[islot, j]],
                                add=True)

            pltpu.sync_copy(srcm.at[pl.ds(row0, ib)], srcbuf.at[0])
            pltpu.sync_copy(dstm.at[pl.ds(row0, ib)], dstbuf.at[0])
            for j in range(D):
                gather(0, j, j % D)

            def blk(b, carry):
                islot = lax.rem(b, 2)
                nslot = lax.rem(b + 1, 2)
                r0 = row0 + (b + 1) * ib
                pltpu.sync_copy(srcm.at[pl.ds(r0, ib)], srcbuf.at[nslot])
                pltpu.sync_copy(dstm.at[pl.ds(r0, ib)], dstbuf.at[nslot])
                for j in range(ib):
                    scatter(islot, j, j % D)
                    jf = (j + D) % ib
                    gather(islot if j + D < ib else nslot, jf, j % D)
                return carry

            lax.fori_loop(0, nblk - 1, blk, 0)

            for j in range(ib):
                scatter(0, j, j % D)
                if j + D < ib:
                    gather(0, j + D, j % D)

            plsc.subcore_barrier()

            @pl.when(sid < NSUB - 1)
            def _wb():
                pltpu.sync_copy(acc.at[pl.ds(sid * wr, wr)],
                                out_hbm.at[pl.ds(sid * wr, wr)])

            @pl.when(sid == NSUB - 1)
            def _wb_last():
                pltpu.sync_copy(acc.at[pl.ds((NSUB - 1) * wr, wr_last)],
                                out_hbm.at[pl.ds((NSUB - 1) * wr, wr_last)])

        for c in range(2):
            @pl.when(cid == c)
            def _core(c=c):
                for s in range(npc):
                    run(hs[c * npc + s], outs[c * npc + s])

    return pl.kernel(
        body,
        out_type=[jax.ShapeDtypeStruct((N, FS), jnp.float32)] * (2 * npc),
        mesh=mesh,
        compiler_params=pltpu.CompilerParams(use_tc_tiling_on_sc=False),
        scratch_types=[
            pltpu.VMEM((2, ib, CHUNK), jnp.int32),
            pltpu.VMEM((2, ib, CHUNK), jnp.int32),
            pltpu.VMEM((4, CHUNK, FS), jnp.float32),
            pltpu.VMEM_SHARED((NACC, FS), jnp.float32),
            pltpu.VMEM_SHARED((NACC, FS), jnp.float32),
            pltpu.SemaphoreType.DMA,
            pltpu.SemaphoreType.DMA,
            pltpu.SemaphoreType.DMA,
            pltpu.SemaphoreType.DMA,
        ],
    )


def kernel(x, edge_index, batch, shape_emb, color_emb, W_node, b_node,
           W1_rel, b1_rel, W1_root, W2_rel, b2_rel, W2_root, W_cls, b_cls):
    x0 = x[:, 0].reshape(GRID, 1, R).astype(jnp.int32)
    x1 = x[:, 1].reshape(GRID, 1, R).astype(jnp.int32)
    batch3 = batch.reshape(GRID, 1, R).astype(jnp.int32)

    src = edge_index[0].astype(jnp.int32)
    dst = edge_index[1].astype(jnp.int32)
    pad = EP - E
    pad_dst = N + (jnp.arange(pad, dtype=jnp.int32) % (NACC - N))
    srcm = jnp.concatenate([src, jnp.zeros((pad,), jnp.int32)]).reshape(EP // CHUNK, CHUNK)
    dstm = jnp.concatenate([dst, pad_dst]).reshape(EP // CHUNK, CHUNK)

    zr = NACC // NSUB
    z16 = jnp.zeros((zr, FS), jnp.float32)

    wn1t = W_node[:, :8].T
    wn2t = W_node[:, 8:].T
    bn = b_node.reshape(1, 32)
    wc1 = jnp.concatenate([W1_rel.T, W1_root.T], axis=0)
    b1 = b1_rel.reshape(1, 64)
    wc2 = jnp.concatenate([W2_rel.T, W2_root.T], axis=0)
    b2 = b2_rel.reshape(1, 64)
    wclst = W_cls.T
    bcls = b_cls.reshape(1, 10)

    h0a, h0b = _node_mlp(x0, x1, shape_emb, color_emb, wn1t, wn2t, bn)
    a1a, a1b = _make_sc_agg(1)(h0a, h0b, srcm, dstm, z16)
    h1 = _conv1_lin(a1a, a1b, h0a, h0b, wc1, b1)
    a2 = _make_sc_agg(2)(*h1, srcm, dstm, z16)
    return _final(a2, h1, wc2, b2, batch3, wclst, bcls)


# speedup vs baseline: 9.8015x; 1.0604x over previous
import functools

import jax
import jax.numpy as jnp
from jax import lax
from jax.experimental import pallas as pl
from jax.experimental.pallas import tpu as pltpu
from jax.experimental.pallas import tpu_sc as plsc

N = 50000
E = 800000
G = 256
EP = 819200
CHUNK = 128
NSUB = 16
NACC = 50048
FS = 16
R = 2000
GRID = N // R


def _node_mlp_body(x0_ref, x1_ref, se_ref, ce_ref, wn1_ref, wn2_ref, b_ref,
                   h0a_ref, h0b_ref):
    tab_s = jnp.dot(se_ref[...], wn1_ref[...], preferred_element_type=jnp.float32,
                    precision=lax.Precision.HIGHEST)
    tab_c = jnp.dot(ce_ref[...], wn2_ref[...], preferred_element_type=jnp.float32,
                    precision=lax.Precision.HIGHEST)
    tab = jnp.concatenate([tab_s, tab_c], axis=0)
    x0 = jnp.reshape(x0_ref[...], (1, R))
    x1 = jnp.reshape(x1_ref[...], (1, R))
    j = lax.broadcasted_iota(jnp.int32, (32, R), 0)
    tgt = jnp.where(j < 16, x0, x1 + 16)
    oht = (j == tgt).astype(jnp.float32)
    h0 = lax.dot_general(oht, tab, (((0,), (0,)), ((), ())),
                         preferred_element_type=jnp.float32,
                         precision=lax.Precision.HIGHEST)
    h0 = jnp.maximum(h0 + b_ref[...], 0.0)
    h0a_ref[...] = h0[:, :16]
    h0b_ref[...] = h0[:, 16:]


def _node_mlp(x0, x1, shape_emb, color_emb, wn1t, wn2t, b_node):
    return pl.pallas_call(
        _node_mlp_body,
        grid=(GRID,),
        in_specs=[
            pl.BlockSpec((1, 1, R), lambda i: (i, 0, 0)),
            pl.BlockSpec((1, 1, R), lambda i: (i, 0, 0)),
            pl.BlockSpec((16, 8), lambda i: (0, 0)),
            pl.BlockSpec((16, 8), lambda i: (0, 0)),
            pl.BlockSpec((8, 32), lambda i: (0, 0)),
            pl.BlockSpec((8, 32), lambda i: (0, 0)),
            pl.BlockSpec((1, 32), lambda i: (0, 0)),
        ],
        out_specs=[
            pl.BlockSpec((R, 16), lambda i: (i, 0)),
            pl.BlockSpec((R, 16), lambda i: (i, 0)),
        ],
        out_shape=[
            jax.ShapeDtypeStruct((N, 16), jnp.float32),
            jax.ShapeDtypeStruct((N, 16), jnp.float32),
        ],
    )(x0, x1, shape_emb, color_emb, wn1t, wn2t, b_node)


def _conv_lin_body(a_ref, b_ref, c_ref, d_ref, wc_ref, bias_ref,
                   o1_ref, o2_ref, o3_ref, o4_ref):
    cat = jnp.concatenate(
        [a_ref[...], b_ref[...], c_ref[...], d_ref[...]], axis=1)
    h = jnp.dot(cat, wc_ref[...], preferred_element_type=jnp.float32,
                precision=lax.Precision.HIGHEST)
    h = jnp.maximum(h + bias_ref[...], 0.0)
    o1_ref[...] = h[:, :16]
    o2_ref[...] = h[:, 16:32]
    o3_ref[...] = h[:, 32:48]
    o4_ref[...] = h[:, 48:]


def _conv1_lin(a1a, a1b, h0a, h0b, wc, bias):
    return pl.pallas_call(
        _conv_lin_body,
        grid=(GRID,),
        in_specs=[
            pl.BlockSpec((R, 16), lambda i: (i, 0)),
            pl.BlockSpec((R, 16), lambda i: (i, 0)),
            pl.BlockSpec((R, 16), lambda i: (i, 0)),
            pl.BlockSpec((R, 16), lambda i: (i, 0)),
            pl.BlockSpec((64, 64), lambda i: (0, 0)),
            pl.BlockSpec((1, 64), lambda i: (0, 0)),
        ],
        out_specs=[pl.BlockSpec((R, 16), lambda i: (i, 0))] * 4,
        out_shape=[jax.ShapeDtypeStruct((N, 16), jnp.float32)] * 4,
    )(a1a, a1b, h0a, h0b, wc, bias)


def _final_body(a_ref, b_ref, c_ref, d_ref, e_ref, f_ref, g_ref, hh_ref,
                wc_ref, bias_ref, batch_ref,
                wcls_ref, bcls_ref, out_ref, sums_ref, cnt_ref):
    i = pl.program_id(0)
    cat = jnp.concatenate(
        [a_ref[...], b_ref[...], c_ref[...], d_ref[...],
         e_ref[...], f_ref[...], g_ref[...], hh_ref[...]], axis=1)
    h2 = jnp.dot(cat, wc_ref[...], preferred_element_type=jnp.float32,
                 precision=lax.Precision.HIGHEST)
    h2 = jnp.maximum(h2 + bias_ref[...], 0.0)
    brow = jnp.reshape(batch_ref[...], (1, R))
    gi = lax.broadcasted_iota(jnp.int32, (G, R), 0)
    oht = (gi == brow).astype(jnp.float32)
    psum = jnp.dot(oht, h2, preferred_element_type=jnp.float32,
                   precision=lax.Precision.HIGHEST)
    pcnt = jnp.sum(oht, axis=1, keepdims=True)

    @pl.when(i == 0)
    def _init():
        sums_ref[...] = psum
        cnt_ref[...] = pcnt

    @pl.when(i > 0)
    def _acc():
        sums_ref[...] += psum
        cnt_ref[...] += pcnt

    @pl.when(i == GRID - 1)
    def _fin():
        logits = jnp.dot(sums_ref[...], wcls_ref[...],
                         preferred_element_type=jnp.float32,
                         precision=lax.Precision.HIGHEST)
        out_ref[...] = logits / jnp.maximum(cnt_ref[...], 1.0) + bcls_ref[...]


def _final(a2, h1, wc, bias, batch3, wclst, bcls):
    return pl.pallas_call(
        _final_body,
        grid=(GRID,),
        in_specs=[pl.BlockSpec((R, 16), lambda i: (i, 0))] * 8 + [
            pl.BlockSpec((128, 64), lambda i: (0, 0)),
            pl.BlockSpec((1, 64), lambda i: (0, 0)),
            pl.BlockSpec((1, 1, R), lambda i: (i, 0, 0)),
            pl.BlockSpec((64, 10), lambda i: (0, 0)),
            pl.BlockSpec((1, 10), lambda i: (0, 0)),
        ],
        out_specs=pl.BlockSpec((G, 10), lambda i: (0, 0)),
        out_shape=jax.ShapeDtypeStruct((G, 10), jnp.float32),
        scratch_shapes=[
            pltpu.VMEM((G, 64), jnp.float32),
            pltpu.VMEM((G, 1), jnp.float32),
        ],
    )(*a2, *h1, wc, bias, batch3, wclst, bcls)


@functools.cache
def _make_sc_agg(npc):
    eps = EP // NSUB
    nchunk = eps // CHUNK
    ib = 16
    nblk = nchunk // ib
    zr = NACC // NSUB
    wr = 3128
    wr_last = N - 15 * wr
    D = 8
    GL = 4

    mesh = plsc.VectorSubcoreMesh(core_axis_name="c", subcore_axis_name="s",
                                  num_cores=2, num_subcores=NSUB)

    def body(*refs):
        hs = refs[:2 * npc]
        srcm, dstm, zeros = refs[2 * npc:2 * npc + 3]
        outs = refs[2 * npc + 3:4 * npc + 3]
        sc = refs[4 * npc + 3:]
        srcbuf, dstbuf, rows, acc, h_sp = sc[:5]
        gsem = sc[5:5 + D]
        ssem = sc[5 + D:5 + 2 * D]
        cid = lax.axis_index("c")
        sid = lax.axis_index("s")

        def run(h_hbm, out_hbm):
            pltpu.sync_copy(zeros, acc.at[pl.ds(sid * zr, zr)])

            @pl.when(sid < NSUB - 1)
            def _ld():
                pltpu.sync_copy(h_hbm.at[pl.ds(sid * wr, wr)],
                                h_sp.at[pl.ds(sid * wr, wr)])

            @pl.when(sid == NSUB - 1)
            def _ld_last():
                pltpu.sync_copy(h_hbm.at[pl.ds((NSUB - 1) * wr, wr_last)],
                                h_sp.at[pl.ds((NSUB - 1) * wr, wr_last)])

            plsc.subcore_barrier()

            row0 = sid * nchunk

            def gather(islot, j, rslot):
                pltpu.async_copy(h_sp.at[srcbuf.at[islot, j]],
                                 rows.at[rslot], gsem[rslot])

            def wait_gather(islot, j, rslot):
                pltpu.make_async_copy(h_sp.at[srcbuf.at[islot, j]],
                                      rows.at[rslot], gsem[rslot]).wait()

            def scatter(islot, j, rslot):
                pltpu.async_copy(rows.at[rslot], acc.at[dstbuf.at[islot, j]],
                                 ssem[rslot], add=True)

            def wait_scatter(rslot):
                pltpu.make_async_copy(rows.at[rslot], acc.at[dstbuf.at[0, 0]],
                                      ssem[rslot]).wait()

            def load_idx(b, slot):
                r0 = row0 + b * ib
                pltpu.sync_copy(srcm.at[pl.ds(r0, ib)], srcbuf.at[slot])
                pltpu.sync_copy(dstm.at[pl.ds(r0, ib)], dstbuf.at[slot])

            load_idx(0, 0)
            for j in range(GL):
                gather(0, j, j)
            load_idx(1, 1)
            for j in range(ib):
                r = j % D
                wait_gather(0, j, r)
                scatter(0, j, r)
                cg = j + GL
                rg = cg % D
                if cg - D >= 0:
                    wait_scatter(rg)
                if cg < ib:
                    gather(0, cg, rg)
                else:
                    gather(1, cg - ib, rg)

            def blk(b, carry):
                islot = lax.rem(b, 2)
                nslot = lax.rem(b + 1, 2)
                load_idx_r0 = row0 + (b + 1) * ib
                pltpu.sync_copy(srcm.at[pl.ds(load_idx_r0, ib)],
                                srcbuf.at[nslot])
                pltpu.sync_copy(dstm.at[pl.ds(load_idx_r0, ib)],
                                dstbuf.at[nslot])
                for j in range(ib):
                    r = j % D
                    wait_gather(islot, j, r)
                    scatter(islot, j, r)
                    cg = j + GL
                    rg = cg % D
                    wait_scatter(rg)
                    if cg < ib:
                        gather(islot, cg, rg)
                    else:
                        gather(nslot, cg - ib, rg)
                return carry

            lax.fori_loop(1, nblk - 1, blk, 0)

            for j in range(ib):
                r = j % D
                wait_gather(0, j, r)
                scatter(0, j, r)
                cg = j + GL
                if cg < ib:
                    rg = cg % D
                    wait_scatter(rg)
                    gather(0, cg, rg)
            for r in range(D):
                wait_scatter(r)

            plsc.subcore_barrier()

            @pl.when(sid < NSUB - 1)
            def _wb():
                pltpu.sync_copy(acc.at[pl.ds(sid * wr, wr)],
                                out_hbm.at[pl.ds(sid * wr, wr)])

            @pl.when(sid == NSUB - 1)
            def _wb_last():
                pltpu.sync_copy(acc.at[pl.ds((NSUB - 1) * wr, wr_last)],
                                out_hbm.at[pl.ds((NSUB - 1) * wr, wr_last)])

        for c in range(2):
            @pl.when(cid == c)
            def _core(c=c):
                for s in range(npc):
                    run(hs[c * npc + s], outs[c * npc + s])

    return pl.kernel(
        body,
        out_type=[jax.ShapeDtypeStruct((N, FS), jnp.float32)] * (2 * npc),
        mesh=mesh,
        compiler_params=pltpu.CompilerParams(use_tc_tiling_on_sc=False),
        scratch_types=[
            pltpu.VMEM((2, ib, CHUNK), jnp.int32),
            pltpu.VMEM((2, ib, CHUNK), jnp.int32),
            pltpu.VMEM((D, CHUNK, FS), jnp.float32),
            pltpu.VMEM_SHARED((NACC, FS), jnp.float32),
            pltpu.VMEM_SHARED((NACC, FS), jnp.float32),
        ] + [pltpu.SemaphoreType.DMA] * (2 * D),
    )


def kernel(x, edge_index, batch, shape_emb, color_emb, W_node, b_node,
           W1_rel, b1_rel, W1_root, W2_rel, b2_rel, W2_root, W_cls, b_cls):
    x0 = x[:, 0].reshape(GRID, 1, R).astype(jnp.int32)
    x1 = x[:, 1].reshape(GRID, 1, R).astype(jnp.int32)
    batch3 = batch.reshape(GRID, 1, R).astype(jnp.int32)

    src = edge_index[0].astype(jnp.int32)
    dst = edge_index[1].astype(jnp.int32)
    pad = EP - E
    pad_dst = N + (jnp.arange(pad, dtype=jnp.int32) % (NACC - N))
    srcm = jnp.concatenate([src, jnp.zeros((pad,), jnp.int32)]).reshape(EP // CHUNK, CHUNK)
    dstm = jnp.concatenate([dst, pad_dst]).reshape(EP // CHUNK, CHUNK)

    zr = NACC // NSUB
    z16 = jnp.zeros((zr, FS), jnp.float32)

    wn1t = W_node[:, :8].T
    wn2t = W_node[:, 8:].T
    bn = b_node.reshape(1, 32)
    wc1 = jnp.concatenate([W1_rel.T, W1_root.T], axis=0)
    b1 = b1_rel.reshape(1, 64)
    wc2 = jnp.concatenate([W2_rel.T, W2_root.T], axis=0)
    b2 = b2_rel.reshape(1, 64)
    wclst = W_cls.T
    bcls = b_cls.reshape(1, 10)

    h0a, h0b = _node_mlp(x0, x1, shape_emb, color_emb, wn1t, wn2t, bn)
    a1a, a1b = _make_sc_agg(1)(h0a, h0b, srcm, dstm, z16)
    h1 = _conv1_lin(a1a, a1b, h0a, h0b, wc1, b1)
    a2 = _make_sc_agg(2)(*h1, srcm, dstm, z16)
    return _final(a2, h1, wc2, b2, batch3, wclst, bcls)
